# baseline (speedup 1.0000x reference)
import functools

import jax
import jax.numpy as jnp
from jax import lax
from jax.experimental import pallas as pl
from jax.experimental.pallas import tpu as pltpu
from jax.experimental.pallas import tpu_sc as plsc

_TOPK = 8
_NCOL = 64
_NROW = 32768
_NWORKERS = 32
_RPW = _NROW // _NWORKERS
_CH = 256
_TCB = 4096


def _noise_block(xt_ref, epst_ref, outt_ref):
    x = xt_ref[...]
    eps = epst_ref[...]
    outt_ref[...] = x + eps * jax.nn.softplus(x)


def _noisy_t(logits):
    eps_t = jax.random.normal(
        jax.random.key(42), logits.shape, dtype=logits.dtype
    ).T
    return pl.pallas_call(
        _noise_block,
        grid=(_NROW // _TCB,),
        in_specs=[
            pl.BlockSpec((_NCOL, _TCB), lambda i: (0, i)),
            pl.BlockSpec((_NCOL, _TCB), lambda i: (0, i)),
        ],
        out_specs=pl.BlockSpec((_NCOL, _TCB), lambda i: (0, i)),
        out_shape=jax.ShapeDtypeStruct((_NCOL, _NROW), logits.dtype),
    )(logits.T, eps_t)


def _sc_body(noisyt_hbm, out_hbm, idxt_hbm, nt_v, out_v, idxt_v):
    wid = lax.axis_index("s") * 2 + lax.axis_index("c")
    lane = lax.broadcasted_iota(jnp.int32, (16,), 0)

    def chunk_body(ch, carry):
        base = wid * _RPW + ch * _CH
        pltpu.sync_copy(noisyt_hbm.at[:, pl.ds(base, _CH)], nt_v)

        def zero_body(i, c):
            out_v[i, pl.ds(0, 16)] = jnp.zeros((16,), jnp.float32)
            out_v[i, pl.ds(16, 16)] = jnp.zeros((16,), jnp.float32)
            out_v[i, pl.ds(32, 16)] = jnp.zeros((16,), jnp.float32)
            out_v[i, pl.ds(48, 16)] = jnp.zeros((16,), jnp.float32)
            return c

        lax.fori_loop(0, _CH, zero_body, 0)

        def group_body(g, c):
            rbase = g * 16
            work = [nt_v[j, pl.ds(rbase, 16)] for j in range(_NCOL)]
            vals = []
            idxs = []
            for k in range(_TOPK):
                tv = list(work)
                ti = list(range(_NCOL))
                first = True
                while len(tv) > 1:
                    nv, ni = [], []
                    for t in range(0, len(tv), 2):
                        cmp = tv[t] >= tv[t + 1]
                        nv.append(jnp.where(cmp, tv[t], tv[t + 1]))
                        if first:
                            ni.append(
                                jnp.where(cmp,
                                          jnp.full((16,), ti[t], jnp.int32),
                                          jnp.full((16,), ti[t + 1],
                                                   jnp.int32)))
                        else:
                            ni.append(jnp.where(cmp, ti[t], ti[t + 1]))
                    tv, ti = nv, ni
                    first = False
                vals.append(tv[0])
                idxs.append(ti[0])
                if k < _TOPK - 1:
                    sel = ti[0]
                    work = [jnp.where(sel == j, -jnp.inf, work[j])
                            for j in range(_NCOL)]
            m0 = vals[0]
            es = [jnp.exp(v - m0) for v in vals]
            s01 = (es[0] + es[1]) + (es[2] + es[3])
            s23 = (es[4] + es[5]) + (es[6] + es[7])
            inv = 1.0 / (s01 + s23)
            rowvec = lane + rbase
            for k in range(_TOPK):
                idxt_v[k, pl.ds(rbase, 16)] = idxs[k]
                plsc.store_scatter(out_v, [rowvec, idxs[k]], es[k] * inv)
            return c

        lax.fori_loop(0, _CH // 16, group_body, 0)
        pltpu.sync_copy(out_v, out_hbm.at[pl.ds(base, _CH)])
        pltpu.sync_copy(idxt_v, idxt_hbm.at[:, pl.ds(base, _CH)])
        return carry

    lax.fori_loop(0, _RPW // _CH, chunk_body, 0)


def kernel(logits):
    noisyt = _noisy_t(logits)
    mesh = plsc.VectorSubcoreMesh(core_axis_name="c", subcore_axis_name="s")
    sc = functools.partial(
        pl.kernel,
        mesh=mesh,
        out_type=[
            jax.ShapeDtypeStruct((_NROW, _NCOL), jnp.float32),
            jax.ShapeDtypeStruct((_TOPK, _NROW), jnp.int32),
        ],
        scratch_types=[
            pltpu.VMEM((_NCOL, _CH), jnp.float32),
            pltpu.VMEM((_CH, _NCOL), jnp.float32),
            pltpu.VMEM((_TOPK, _CH), jnp.int32),
        ],
        compiler_params=pltpu.CompilerParams(needs_layout_passes=False),
    )(_sc_body)
    out, idx_t = sc(noisyt)
    return out, idx_t.T


# speedup vs baseline: 1.3170x; 1.3170x over previous
import functools

import jax
import jax.numpy as jnp
from jax import lax
from jax.experimental import pallas as pl
from jax.experimental.pallas import tpu as pltpu
from jax.experimental.pallas import tpu_sc as plsc

_TOPK = 8
_NCOL = 64
_NROW = 32768
_NWORKERS = 32
_SC_ROWS = 12288
_TC_ROWS = _NROW - _SC_ROWS
_RPW = _SC_ROWS // _NWORKERS
_CH = 128
_TCB = 4096
_RTB = 1024


def _noise_block(xt_ref, epst_ref, outt_ref):
    x = xt_ref[...]
    eps = epst_ref[...]
    outt_ref[...] = x + eps * jax.nn.softplus(x)


def _noisy_t(logits):
    eps_t = jax.random.normal(
        jax.random.key(42), logits.shape, dtype=logits.dtype
    ).T
    return pl.pallas_call(
        _noise_block,
        grid=(_NROW // _TCB,),
        in_specs=[
            pl.BlockSpec((_NCOL, _TCB), lambda i: (0, i)),
            pl.BlockSpec((_NCOL, _TCB), lambda i: (0, i)),
        ],
        out_specs=pl.BlockSpec((_NCOL, _TCB), lambda i: (0, i)),
        out_shape=jax.ShapeDtypeStruct((_NCOL, _NROW), logits.dtype),
    )(logits.T, eps_t)


def _tc_router_block(nt_ref, outt_ref, idxt_ref):
    noisy = nt_ref[...]
    rows = jax.lax.broadcasted_iota(jnp.int32, noisy.shape, 0).astype(
        jnp.float32)
    work = noisy
    vals = []
    idxs = []
    for _ in range(_TOPK):
        m = jnp.max(work, axis=0, keepdims=True)
        sel = jnp.min(jnp.where(work == m, rows, float(_NCOL)), axis=0,
                      keepdims=True)
        vals.append(m)
        idxs.append(sel)
        work = jnp.where(rows == sel, -jnp.inf, work)
    v = jnp.concatenate(vals, axis=0)
    fi = jnp.concatenate(idxs, axis=0)
    p = jnp.exp(v - v[0:1])
    p = p / jnp.sum(p, axis=0, keepdims=True)
    out = jnp.zeros_like(noisy)
    for k in range(_TOPK):
        out = jnp.where(rows == fi[k : k + 1], p[k : k + 1], out)
    outt_ref[...] = out
    idxt_ref[...] = fi.astype(jnp.int32)


def _tc_router(noisyt_tail):
    return pl.pallas_call(
        _tc_router_block,
        grid=(_TC_ROWS // _RTB,),
        in_specs=[pl.BlockSpec((_NCOL, _RTB), lambda i: (0, i))],
        out_specs=[
            pl.BlockSpec((_NCOL, _RTB), lambda i: (0, i)),
            pl.BlockSpec((_TOPK, _RTB), lambda i: (0, i)),
        ],
        out_shape=[
            jax.ShapeDtypeStruct((_NCOL, _TC_ROWS), jnp.float32),
            jax.ShapeDtypeStruct((_TOPK, _TC_ROWS), jnp.int32),
        ],
    )(noisyt_tail)


def _sc_body(noisyt_hbm, out_hbm, idxt_hbm, nt_v, out_v, idxt_v):
    wid = lax.axis_index("s") * 2 + lax.axis_index("c")
    lane = lax.broadcasted_iota(jnp.int32, (16,), 0)

    def chunk_body(ch, carry):
        base = wid * _RPW + ch * _CH
        pltpu.sync_copy(noisyt_hbm.at[:, pl.ds(base, _CH)], nt_v)

        def zero_body(i, c):
            out_v[i, pl.ds(0, 16)] = jnp.zeros((16,), jnp.float32)
            out_v[i, pl.ds(16, 16)] = jnp.zeros((16,), jnp.float32)
            out_v[i, pl.ds(32, 16)] = jnp.zeros((16,), jnp.float32)
            out_v[i, pl.ds(48, 16)] = jnp.zeros((16,), jnp.float32)
            return c

        lax.fori_loop(0, _CH, zero_body, 0)

        def group_body(g, c):
            rbase = g * 16
            work = [nt_v[j, pl.ds(rbase, 16)] for j in range(_NCOL)]
            vals = []
            idxs = []
            for k in range(_TOPK):
                tv = list(work)
                ti = list(range(_NCOL))
                first = True
                while len(tv) > 1:
                    nv, ni = [], []
                    for t in range(0, len(tv), 2):
                        cmp = tv[t] >= tv[t + 1]
                        nv.append(jnp.where(cmp, tv[t], tv[t + 1]))
                        if first:
                            ni.append(
                                jnp.where(cmp,
                                          jnp.full((16,), ti[t], jnp.int32),
                                          jnp.full((16,), ti[t + 1],
                                                   jnp.int32)))
                        else:
                            ni.append(jnp.where(cmp, ti[t], ti[t + 1]))
                    tv, ti = nv, ni
                    first = False
                vals.append(tv[0])
                idxs.append(ti[0])
                if k < _TOPK - 1:
                    sel = ti[0]
                    work = [jnp.where(sel == j, -jnp.inf, work[j])
                            for j in range(_NCOL)]
            m0 = vals[0]
            es = [jnp.exp(v - m0) for v in vals]
            s01 = (es[0] + es[1]) + (es[2] + es[3])
            s23 = (es[4] + es[5]) + (es[6] + es[7])
            inv = 1.0 / (s01 + s23)
            rowvec = lane + rbase
            for k in range(_TOPK):
                idxt_v[k, pl.ds(rbase, 16)] = idxs[k]
                plsc.store_scatter(out_v, [rowvec, idxs[k]], es[k] * inv)
            return c

        lax.fori_loop(0, _CH // 16, group_body, 0)
        pltpu.sync_copy(out_v, out_hbm.at[pl.ds(base, _CH)])
        pltpu.sync_copy(idxt_v, idxt_hbm.at[:, pl.ds(base, _CH)])
        return carry

    lax.fori_loop(0, _RPW // _CH, chunk_body, 0)


def kernel(logits):
    noisyt = _noisy_t(logits)
    mesh = plsc.VectorSubcoreMesh(core_axis_name="c", subcore_axis_name="s")
    sc = functools.partial(
        pl.kernel,
        mesh=mesh,
        out_type=[
            jax.ShapeDtypeStruct((_SC_ROWS, _NCOL), jnp.float32),
            jax.ShapeDtypeStruct((_TOPK, _SC_ROWS), jnp.int32),
        ],
        scratch_types=[
            pltpu.VMEM((_NCOL, _CH), jnp.float32),
            pltpu.VMEM((_CH, _NCOL), jnp.float32),
            pltpu.VMEM((_TOPK, _CH), jnp.int32),
        ],
        compiler_params=pltpu.CompilerParams(needs_layout_passes=False),
    )(_sc_body)
    out_sc, idxt_sc = sc(noisyt[:, : _SC_ROWS])
    outt_tc, idxt_tc = _tc_router(noisyt[:, _SC_ROWS :])
    router = jnp.concatenate([out_sc, outt_tc.T], axis=0)
    indices = jnp.concatenate([idxt_sc.T, idxt_tc.T], axis=0)
    return router, indices


# speedup vs baseline: 2.1014x; 1.5956x over previous
import jax
import jax.numpy as jnp
from jax.experimental import pallas as pl

_TOPK = 8
_NCOL = 64
_NROW = 32768
_BLOCK = 2048


def _router_block(xt_ref, epst_ref, outt_ref, idxt_ref):
    x = xt_ref[...]
    eps = epst_ref[...]
    noisy = x + eps * jax.nn.softplus(x)
    rows = jax.lax.broadcasted_iota(jnp.int32, noisy.shape, 0).astype(
        jnp.float32)
    work = noisy
    vals = []
    idxs = []
    for _ in range(_TOPK):
        m = jnp.max(work, axis=0, keepdims=True)
        sel = jnp.min(jnp.where(work == m, rows, float(_NCOL)), axis=0,
                      keepdims=True)
        vals.append(m)
        idxs.append(sel)
        work = jnp.where(rows == sel, -jnp.inf, work)
    v = jnp.concatenate(vals, axis=0)
    fi = jnp.concatenate(idxs, axis=0)
    p = jnp.exp(v - v[0:1])
    p = p / jnp.sum(p, axis=0, keepdims=True)
    out = jnp.zeros_like(x)
    for k in range(_TOPK):
        out = jnp.where(rows == fi[k : k + 1], p[k : k + 1], out)
    outt_ref[...] = out
    idxt_ref[...] = fi.astype(jnp.int32)


def kernel(logits):
    eps_t = jax.random.normal(
        jax.random.key(42), logits.shape, dtype=logits.dtype
    ).T
    xt = logits.T
    grid = (_NROW // _BLOCK,)
    router_t, idx_t = pl.pallas_call(
        _router_block,
        grid=grid,
        in_specs=[
            pl.BlockSpec((_NCOL, _BLOCK), lambda i: (0, i)),
            pl.BlockSpec((_NCOL, _BLOCK), lambda i: (0, i)),
        ],
        out_specs=[
            pl.BlockSpec((_NCOL, _BLOCK), lambda i: (0, i)),
            pl.BlockSpec((_TOPK, _BLOCK), lambda i: (0, i)),
        ],
        out_shape=[
            jax.ShapeDtypeStruct((_NCOL, _NROW), logits.dtype),
            jax.ShapeDtypeStruct((_TOPK, _NROW), jnp.int32),
        ],
    )(xt, eps_t)
    return router_t.T, idx_t.T
